# baseline (speedup 1.0000x reference)
import functools

import jax
import jax.numpy as jnp
from jax.experimental import pallas as pl
from jax.experimental.pallas import tpu as pltpu

_SCORE_THR = 0.3
_IOU_THR = 0.5
_B = 2048


def _nms_body(ct_ref, out_ref, sup_ref):
    t_tiles = ct_ref.shape[0]
    b = ct_ref.shape[2]
    ri = jax.lax.broadcasted_iota(jnp.int32, (b, b), 0)
    ci = jax.lax.broadcasted_iota(jnp.int32, (b, b), 1)
    thrm = jnp.where(ri < ci, _IOU_THR, jnp.inf)
    eye = jnp.where(ri == ci, 1.0, 0.0).astype(jnp.float32)
    sup_ref[...] = jnp.zeros((t_tiles, 1, b), jnp.float32)
    out_ref[...] = jnp.zeros((t_tiles, 1, b), jnp.float32)
    nv = jnp.sum(ct_ref[:, 4, :]).astype(jnp.int32)
    t_used = (nv + b - 1) // b

    def outer(i, _):
        blk = ct_ref[i]
        x1r, y1r, x2r, y2r, vr = (blk[0:1], blk[1:2], blk[2:3], blk[3:4],
                                  blk[4:5])

        def tocol(v):
            return jax.lax.dot_general(
                eye, v, (((1,), (1,)), ((), ())),
                preferred_element_type=jnp.float32)

        x1c, y1c, x2c, y2c = tocol(x1r), tocol(y1r), tocol(x2r), tocol(y2r)
        ac = (x2c - x1c) * (y2c - y1c)

        def iou_vs(rx1, ry1, rx2, ry2, ra):
            ix1 = jnp.maximum(x1c, rx1)
            iy1 = jnp.maximum(y1c, ry1)
            ix2 = jnp.minimum(x2c, rx2)
            iy2 = jnp.minimum(y2c, ry2)
            inter = jnp.maximum(ix2 - ix1, 0.0) * (iy2 - iy1)
            return inter / jnp.maximum(ac + ra - inter, 1e-8)

        ar = (x2r - x1r) * (y2r - y1r)
        s_self = (iou_vs(x1r, y1r, x2r, y2r, ar) > thrm).astype(jnp.float32)
        alive = vr * (1.0 - sup_ref[i])

        def f_cond(c):
            return c[1]

        def f_body(c):
            k = c[0]
            cnt = jax.lax.dot_general(
                k, s_self, (((1,), (0,)), ((), ())),
                preferred_element_type=jnp.float32)
            knew = alive * jnp.where(cnt > 0.5, 0.0, 1.0)
            return (knew, jnp.any(knew != k))

        k = jax.lax.while_loop(f_cond, f_body, (alive, jnp.array(True)))[0]
        out_ref[i] = k

        def inner(j, _):
            blkj = ct_ref[j]
            jx1, jy1, jx2, jy2 = (blkj[0:1], blkj[1:2], blkj[2:3], blkj[3:4])
            ja = (jx2 - jx1) * (jy2 - jy1)
            s_ij = (iou_vs(jx1, jy1, jx2, jy2, ja) > _IOU_THR).astype(
                jnp.float32)
            cnt = jax.lax.dot_general(
                k, s_ij, (((1,), (0,)), ((), ())),
                preferred_element_type=jnp.float32)
            sup_ref[j] = jnp.maximum(sup_ref[j],
                                     jnp.where(cnt > 0.5, 1.0, 0.0))
            return 0

        jax.lax.fori_loop(i + 1, t_used, inner, 0)
        return 0

    jax.lax.fori_loop(0, t_used, outer, 0)


@jax.jit
def kernel(boxes, scores):
    boxes = boxes.reshape(-1, 7)
    scores = scores.reshape(-1)
    n = boxes.shape[0]
    t_tiles = -(-n // _B)
    npad = t_tiles * _B

    sig = jax.nn.sigmoid(scores)
    valid = sig > _SCORE_THR
    eff = jnp.where(valid, sig, -1.0)
    order = jnp.argsort(-eff)

    cu = boxes[:, 0]
    cv = boxes[:, 2]
    half_l = boxes[:, 5] / 2.0
    half_w = boxes[:, 4] / 2.0
    coords = jnp.stack([cu - half_l, cv - half_w, cu + half_l, cv + half_w,
                        valid.astype(jnp.float32)], 0)
    cs = jnp.pad(coords[:, order], ((0, 3), (0, npad - n)))
    ct = cs.reshape(8, t_tiles, _B).transpose(1, 0, 2)

    keep_blocks = pl.pallas_call(
        _nms_body,
        out_shape=jax.ShapeDtypeStruct((t_tiles, 1, _B), jnp.float32),
        scratch_shapes=[pltpu.VMEM((t_tiles, 1, _B), jnp.float32)],
    )(ct)

    keep_sorted = keep_blocks.reshape(npad)[:n]
    keep = jnp.zeros((n,), jnp.float32).at[order].set(keep_sorted)
    out = jnp.concatenate([boxes * keep[:, None], (sig * keep)[:, None]],
                          axis=1)
    return out


# speedup vs baseline: 1.0743x; 1.0743x over previous
import jax
import jax.numpy as jnp
from jax.experimental import pallas as pl
from jax.experimental.pallas import tpu as pltpu

_SCORE_THR = 0.3
_IOU_THR = 0.5
_B = 1024


def _nms_body(ct_ref, out_ref, sup_ref):
    t_tiles = ct_ref.shape[0]
    b = ct_ref.shape[2]
    ri = jax.lax.broadcasted_iota(jnp.int32, (b, b), 0)
    ci = jax.lax.broadcasted_iota(jnp.int32, (b, b), 1)
    thrm = jnp.where(ri < ci, _IOU_THR, jnp.inf)
    eye = jnp.where(ri == ci, 1.0, 0.0).astype(jnp.float32)
    sup_ref[...] = jnp.zeros((t_tiles, 1, b), jnp.float32)
    out_ref[...] = jnp.zeros((t_tiles, 1, b), jnp.float32)
    nv = jnp.sum(ct_ref[:, 4, :]).astype(jnp.int32)
    t_used = (nv + b - 1) // b

    def outer(i, _):
        blk = ct_ref[i]
        x1r, y1r, x2r, y2r, vr = (blk[0:1], blk[1:2], blk[2:3], blk[3:4],
                                  blk[4:5])

        def tocol(v):
            return jax.lax.dot_general(
                eye, v, (((1,), (1,)), ((), ())),
                preferred_element_type=jnp.float32)

        x1c, y1c, x2c, y2c = tocol(x1r), tocol(y1r), tocol(x2r), tocol(y2r)
        ac = (x2c - x1c) * (y2c - y1c)

        def iou_vs(rx1, ry1, rx2, ry2, ra):
            ix1 = jnp.maximum(x1c, rx1)
            iy1 = jnp.maximum(y1c, ry1)
            ix2 = jnp.minimum(x2c, rx2)
            iy2 = jnp.minimum(y2c, ry2)
            inter = jnp.maximum(ix2 - ix1, 0.0) * (iy2 - iy1)
            return inter / jnp.maximum(ac + ra - inter, 1e-8)

        ar = (x2r - x1r) * (y2r - y1r)
        s_self = (iou_vs(x1r, y1r, x2r, y2r, ar) > thrm).astype(jnp.float32)
        alive = vr * (1.0 - sup_ref[i])

        def f_cond(c):
            return c[1]

        def f_body(c):
            k = c[0]
            cnt = jax.lax.dot_general(
                k, s_self, (((1,), (0,)), ((), ())),
                preferred_element_type=jnp.float32)
            knew = alive * jnp.where(cnt > 0.5, 0.0, 1.0)
            return (knew, jnp.any(knew != k))

        k = jax.lax.while_loop(f_cond, f_body, (alive, jnp.array(True)))[0]
        out_ref[i] = k

        def inner(j, _):
            blkj = ct_ref[j]
            jx1, jy1, jx2, jy2 = (blkj[0:1], blkj[1:2], blkj[2:3], blkj[3:4])
            ja = (jx2 - jx1) * (jy2 - jy1)
            s_ij = (iou_vs(jx1, jy1, jx2, jy2, ja) > _IOU_THR).astype(
                jnp.float32)
            cnt = jax.lax.dot_general(
                k, s_ij, (((1,), (0,)), ((), ())),
                preferred_element_type=jnp.float32)
            sup_ref[j] = jnp.maximum(sup_ref[j],
                                     jnp.where(cnt > 0.5, 1.0, 0.0))
            return 0

        jax.lax.fori_loop(i + 1, t_used, inner, 0)
        return 0

    jax.lax.fori_loop(0, t_used, outer, 0)


@jax.jit
def kernel(boxes, scores):
    boxes = boxes.reshape(-1, 7)
    scores = scores.reshape(-1)
    n = boxes.shape[0]
    t_tiles = -(-n // _B)
    npad = t_tiles * _B

    sig = jax.nn.sigmoid(scores)
    valid = sig > _SCORE_THR
    eff = jnp.where(valid, sig, -1.0)
    order = jnp.argsort(-eff)

    cu = boxes[:, 0]
    cv = boxes[:, 2]
    half_l = boxes[:, 5] / 2.0
    half_w = boxes[:, 4] / 2.0
    coords = jnp.stack([cu - half_l, cv - half_w, cu + half_l, cv + half_w,
                        valid.astype(jnp.float32)], 0)
    cs = jnp.pad(coords[:, order], ((0, 3), (0, npad - n)))
    ct = cs.reshape(8, t_tiles, _B).transpose(1, 0, 2)

    keep_blocks = pl.pallas_call(
        _nms_body,
        out_shape=jax.ShapeDtypeStruct((t_tiles, 1, _B), jnp.float32),
        scratch_shapes=[pltpu.VMEM((t_tiles, 1, _B), jnp.float32)],
    )(ct)

    keep_sorted = keep_blocks.reshape(npad)[:n]
    keep = jnp.zeros((n,), jnp.float32).at[order].set(keep_sorted)
    out = jnp.concatenate([boxes * keep[:, None], (sig * keep)[:, None]],
                          axis=1)
    return out
